# Initial kernel scaffold; baseline (speedup 1.0000x reference)
import jax
import jax.numpy as jnp
from jax.experimental import pallas as pl


def kernel(x, edge_index, batch, node_to_subgraph, subgraph_to_graph, W1, b1, g1, be1, W2, b2, g2, be2, eps, lin1_W, lin1_b, bn_g, bn_b, lin2_W, lin2_b):
    raise NotImplementedError("write your pallas kernel here")


# speedup vs baseline: 8.5009x; 8.5009x over previous
import functools

import jax
import jax.numpy as jnp
from jax import lax
from jax.experimental import pallas as pl
from jax.experimental.pallas import tpu as pltpu
from jax.experimental.pallas import tpu_sc as plsc

NC = 2
NS = 16
NW = NC * NS

N = 10000
E = 320000
D = 128
L = 3
S = 2000
G = 64

NP = 10240
NA = 10112
EPB = 64
ECH = 160
IB = 40
NBLK = ECH // IB
EP = NW * ECH * EPB
SA = 2048
PB = 40
PCH = NP // (NW * PB)

_mesh = plsc.VectorSubcoreMesh(
    core_axis_name="c", subcore_axis_name="s", num_cores=NC, num_subcores=NS)


def _edge_body(h_hbm, src_hbm, dst_hbm, n2s_hbm, z_hbm, agg_hbm, sub_hbm,
               src_v, dst_v, rows_a, rows_b, pidx_v, prows,
               acc_e, acc_p, sem_a, sem_b):
    cid = lax.axis_index("c")
    sid = lax.axis_index("s")
    wid = sid * NC + cid
    ez = NA // NS
    pz = SA // NS
    pltpu.sync_copy(z_hbm.at[pl.ds(sid * ez, ez)], acc_e.at[pl.ds(sid * ez, ez)])
    pltpu.sync_copy(z_hbm.at[pl.ds(sid * pz, pz)], acc_p.at[pl.ds(sid * pz, pz)])
    pltpu.sync_copy(n2s_hbm.at[pl.ds(wid * PCH, PCH)], pidx_v)
    plsc.subcore_barrier()
    for c in range(PCH):
        pltpu.sync_copy(h_hbm.at[pl.ds(wid * (PCH * PB) + c * PB, PB)], prows)
        pltpu.sync_copy(prows, acc_p.at[pidx_v.at[c]], add=True)
    bufs = (rows_a, rows_b)
    sems = (sem_a, sem_b)
    for blk in range(NBLK):
        pltpu.sync_copy(src_hbm.at[pl.ds(wid * ECH + blk * IB, IB)], src_v)
        pltpu.sync_copy(dst_hbm.at[pl.ds(wid * ECH + blk * IB, IB)], dst_v)
        descs = [pltpu.async_copy(h_hbm.at[src_v.at[0]], rows_a, sem_a)]
        for c in range(IB):
            if c + 1 < IB:
                descs.append(pltpu.async_copy(
                    h_hbm.at[src_v.at[c + 1]], bufs[(c + 1) % 2],
                    sems[(c + 1) % 2]))
            descs[c].wait()
            pltpu.sync_copy(bufs[c % 2], acc_e.at[dst_v.at[c]], add=True)
    plsc.subcore_barrier()
    pltpu.sync_copy(acc_e.at[pl.ds(sid * ez, ez)],
                    agg_hbm.at[cid, pl.ds(sid * ez, ez)])
    pltpu.sync_copy(acc_p.at[pl.ds(sid * pz, pz)],
                    sub_hbm.at[cid, pl.ds(sid * pz, pz)])


_edge_call = functools.partial(
    pl.kernel,
    out_type=(jax.ShapeDtypeStruct((NC, NA, D), jnp.float32),
              jax.ShapeDtypeStruct((NC, SA, D), jnp.float32)),
    mesh=_mesh,
    scratch_types=[
        pltpu.VMEM((IB, EPB), jnp.int32),
        pltpu.VMEM((IB, EPB), jnp.int32),
        pltpu.VMEM((EPB, D), jnp.float32),
        pltpu.VMEM((EPB, D), jnp.float32),
        pltpu.VMEM((PCH, PB), jnp.int32),
        pltpu.VMEM((PB, D), jnp.float32),
        pltpu.VMEM_SHARED((NA, D), jnp.float32),
        pltpu.VMEM_SHARED((SA, D), jnp.float32),
        pltpu.SemaphoreType.DMA,
        pltpu.SemaphoreType.DMA,
    ],
)(_edge_body)


def _pool_body(h_hbm, n2s_hbm, z_hbm, sub_hbm, pidx_v, prows, acc_p, sem):
    cid = lax.axis_index("c")
    sid = lax.axis_index("s")
    wid = sid * NC + cid
    pz = SA // NS
    pltpu.sync_copy(z_hbm.at[pl.ds(sid * pz, pz)], acc_p.at[pl.ds(sid * pz, pz)])
    pltpu.sync_copy(n2s_hbm.at[pl.ds(wid * PCH, PCH)], pidx_v)
    plsc.subcore_barrier()
    for c in range(PCH):
        pltpu.sync_copy(h_hbm.at[pl.ds(wid * (PCH * PB) + c * PB, PB)], prows)
        pltpu.sync_copy(prows, acc_p.at[pidx_v.at[c]], add=True)
    plsc.subcore_barrier()
    pltpu.sync_copy(acc_p.at[pl.ds(sid * pz, pz)],
                    sub_hbm.at[cid, pl.ds(sid * pz, pz)])


_pool_call = functools.partial(
    pl.kernel,
    out_type=jax.ShapeDtypeStruct((NC, SA, D), jnp.float32),
    mesh=_mesh,
    scratch_types=[
        pltpu.VMEM((PCH, PB), jnp.int32),
        pltpu.VMEM((PB, D), jnp.float32),
        pltpu.VMEM_SHARED((SA, D), jnp.float32),
        pltpu.SemaphoreType.DMA,
    ],
)(_pool_body)


def _mlp_body(h_ref, agg_ref, eps_ref, W1_ref, b1_ref, g1_ref, be1_ref,
              W2_ref, b2_ref, g2_ref, be2_ref, out_ref):
    h = h_ref[pl.ds(0, N), :]
    a = agg_ref[0, pl.ds(0, N), :] + agg_ref[1, pl.ds(0, N), :]
    z = (1.0 + eps_ref[0, 0]) * h + a
    z = jnp.dot(z, W1_ref[...], preferred_element_type=jnp.float32) + b1_ref[...]
    m = jnp.mean(z, axis=0, keepdims=True)
    v = jnp.mean((z - m) * (z - m), axis=0, keepdims=True)
    z = (z - m) / jnp.sqrt(v + 1e-5) * g1_ref[...] + be1_ref[...]
    z = jnp.maximum(z, 0.0)
    z = jnp.dot(z, W2_ref[...], preferred_element_type=jnp.float32) + b2_ref[...]
    m = jnp.mean(z, axis=0, keepdims=True)
    v = jnp.mean((z - m) * (z - m), axis=0, keepdims=True)
    z = (z - m) / jnp.sqrt(v + 1e-5) * g2_ref[...] + be2_ref[...]
    z = jnp.maximum(z, 0.0)
    out_ref[pl.ds(0, N), :] = z
    out_ref[pl.ds(N, NP - N), :] = jnp.zeros((NP - N, D), jnp.float32)


_mlp_call = pl.pallas_call(
    _mlp_body,
    out_shape=jax.ShapeDtypeStruct((NP, D), jnp.float32),
)


def _head_body(sub1_ref, sub2_ref, sub3_ref, s2g_ref, lin1W_ref, lin1b_ref,
               bng_ref, bnb_ref, lin2W_ref, lin2b_ref, out_ref):
    s2g = s2g_ref[...]
    oh = (lax.broadcasted_iota(jnp.int32, (G, SA), 0) == s2g).astype(jnp.float32)
    cnt = jnp.sum(oh, axis=1, keepdims=True)
    cnt = jnp.maximum(cnt, 1.0)
    t = lin1b_ref[...]
    for l, ref in enumerate((sub1_ref, sub2_ref, sub3_ref)):
        sub_l = ref[0] + ref[1]
        gs_l = jnp.dot(oh, sub_l, preferred_element_type=jnp.float32) / cnt
        t = t + jnp.dot(gs_l, lin1W_ref[pl.ds(l * D, D), :],
                        preferred_element_type=jnp.float32)
    m = jnp.mean(t, axis=0, keepdims=True)
    v = jnp.mean((t - m) * (t - m), axis=0, keepdims=True)
    t = (t - m) / jnp.sqrt(v + 1e-5) * bng_ref[...] + bnb_ref[...]
    t = jnp.maximum(t, 0.0)
    o = jnp.dot(t, lin2W_ref[...], preferred_element_type=jnp.float32) + lin2b_ref[...]
    mx = jnp.max(o, axis=1, keepdims=True)
    lse = jnp.log(jnp.sum(jnp.exp(o - mx), axis=1, keepdims=True))
    out_ref[...] = o - mx - lse


def kernel(x, edge_index, batch, node_to_subgraph, subgraph_to_graph,
           W1, b1, g1, be1, W2, b2, g2, be2, eps,
           lin1_W, lin1_b, bn_g, bn_b, lin2_W, lin2_b):
    C = lin2_W.shape[1]
    f32 = jnp.float32

    src = edge_index[0].astype(jnp.int32)
    dst = edge_index[1].astype(jnp.int32)
    npad = EP - E
    pad_src = N + (jnp.arange(npad, dtype=jnp.int32) % (NP - N))
    pad_dst = N + (jnp.arange(npad, dtype=jnp.int32) % (NA - N))
    src2d = jnp.concatenate([src, pad_src]).reshape(NW * ECH, EPB)
    dst2d = jnp.concatenate([dst, pad_dst]).reshape(NW * ECH, EPB)

    n2s = node_to_subgraph.astype(jnp.int32)
    pad_n2s = jnp.zeros((NP - N,), dtype=jnp.int32)
    n2s2d = jnp.concatenate([n2s, pad_n2s]).reshape(NW * PCH, PB)

    s2g_pad = jnp.concatenate(
        [subgraph_to_graph.astype(jnp.int32),
         jnp.full((SA - S,), G, dtype=jnp.int32)]).reshape(1, SA)

    zeros_np = jnp.zeros((NP, D), f32)
    h = jnp.concatenate([x, jnp.zeros((NP - N, D), f32)], axis=0)

    subs = []
    for l in range(L):
        aggp, subp = _edge_call(h, src2d, dst2d, n2s2d, zeros_np)
        if l > 0:
            subs.append(subp)
        h = _mlp_call(h, aggp, (1.0 * eps[l]).reshape(1, 1),
                      W1[l], b1[l].reshape(1, D), g1[l].reshape(1, D),
                      be1[l].reshape(1, D),
                      W2[l], b2[l].reshape(1, D), g2[l].reshape(1, D),
                      be2[l].reshape(1, D))
    subs.append(_pool_call(h, n2s2d, zeros_np))

    head = pl.pallas_call(
        _head_body,
        out_shape=jax.ShapeDtypeStruct((G, C), f32),
    )
    return head(subs[0], subs[1], subs[2], s2g_pad,
                lin1_W, lin1_b.reshape(1, D), bn_g.reshape(1, D),
                bn_b.reshape(1, D), lin2_W, lin2_b.reshape(1, C))


# speedup vs baseline: 10.5639x; 1.2427x over previous
import functools

import jax
import jax.numpy as jnp
from jax import lax
from jax.experimental import pallas as pl
from jax.experimental.pallas import tpu as pltpu
from jax.experimental.pallas import tpu_sc as plsc

NC = 2
NS = 16
NW = NC * NS

N = 10000
E = 320000
D = 128
L = 3
S = 2000
G = 64

NP = 10240
NA = 10112
EPB = 64
ECH = 160
IB = 16
NBLK = ECH // IB
NBUF = 4
LAG = 2
EP = NW * ECH * EPB
SA = 2048
PB = 40
PCH = NP // (NW * PB)

_mesh = plsc.VectorSubcoreMesh(
    core_axis_name="c", subcore_axis_name="s", num_cores=NC, num_subcores=NS)


def _edge_body(h_hbm, src_hbm, dst_hbm, z_hbm, agg_hbm,
               src0, src1, dst0, dst1, r0, r1, r2, r3,
               acc_e, g0, g1, g2, g3, s0, s1, s2, s3, isem):
    cid = lax.axis_index("c")
    sid = lax.axis_index("s")
    wid = sid * NC + cid
    ez = NA // NS
    pltpu.sync_copy(z_hbm.at[pl.ds(sid * ez, ez)], acc_e.at[pl.ds(sid * ez, ez)])
    plsc.subcore_barrier()
    srcs = (src0, src1)
    dsts = (dst0, dst1)
    bufs = (r0, r1, r2, r3)
    gsems = (g0, g1, g2, g3)
    ssems = (s0, s1, s2, s3)
    gdesc = [None] * ECH
    sdesc = [None] * ECH
    idesc = [None, None]

    def _scatter(u):
        ub, uj = divmod(u, IB)
        gdesc[u].wait()
        sdesc[u] = pltpu.async_copy(
            bufs[u % NBUF], acc_e.at[dsts[ub % 2].at[uj]],
            ssems[u % NBUF], add=True)

    pltpu.sync_copy(src_hbm.at[pl.ds(wid * ECH, IB)], srcs[0])
    pltpu.sync_copy(dst_hbm.at[pl.ds(wid * ECH, IB)], dsts[0])
    for t in range(ECH):
        blk, j = divmod(t, IB)
        if j == 0 and blk > 0:
            idesc[0].wait()
            idesc[1].wait()
        if j == NBUF and blk + 1 < NBLK:
            nb = blk + 1
            idesc[0] = pltpu.async_copy(
                src_hbm.at[pl.ds(wid * ECH + nb * IB, IB)], srcs[nb % 2], isem)
            idesc[1] = pltpu.async_copy(
                dst_hbm.at[pl.ds(wid * ECH + nb * IB, IB)], dsts[nb % 2], isem)
        if t >= NBUF:
            sdesc[t - NBUF].wait()
        gdesc[t] = pltpu.async_copy(
            h_hbm.at[srcs[blk % 2].at[j]], bufs[t % NBUF], gsems[t % NBUF])
        if t >= LAG:
            _scatter(t - LAG)
    for u in range(ECH - LAG, ECH):
        _scatter(u)
    for u in range(ECH - NBUF, ECH):
        sdesc[u].wait()
    plsc.subcore_barrier()
    pltpu.sync_copy(acc_e.at[pl.ds(sid * ez, ez)],
                    agg_hbm.at[cid, pl.ds(sid * ez, ez)])


_edge_call = functools.partial(
    pl.kernel,
    out_type=jax.ShapeDtypeStruct((NC, NA, D), jnp.float32),
    mesh=_mesh,
    scratch_types=[
        pltpu.VMEM((IB, EPB), jnp.int32),
        pltpu.VMEM((IB, EPB), jnp.int32),
        pltpu.VMEM((IB, EPB), jnp.int32),
        pltpu.VMEM((IB, EPB), jnp.int32),
        pltpu.VMEM((EPB, D), jnp.float32),
        pltpu.VMEM((EPB, D), jnp.float32),
        pltpu.VMEM((EPB, D), jnp.float32),
        pltpu.VMEM((EPB, D), jnp.float32),
        pltpu.VMEM_SHARED((NA, D), jnp.float32),
        pltpu.SemaphoreType.DMA,
        pltpu.SemaphoreType.DMA,
        pltpu.SemaphoreType.DMA,
        pltpu.SemaphoreType.DMA,
        pltpu.SemaphoreType.DMA,
        pltpu.SemaphoreType.DMA,
        pltpu.SemaphoreType.DMA,
        pltpu.SemaphoreType.DMA,
        pltpu.SemaphoreType.DMA,
    ],
)(_edge_body)


def _pool_body(h1, h2, h3, n2s_hbm, z_hbm, sub_hbm,
               pidx_v, prows, acc0, acc1, acc2, sem):
    cid = lax.axis_index("c")
    sid = lax.axis_index("s")
    wid = sid * NC + cid
    pz = SA // NS
    accs = (acc0, acc1, acc2)
    for acc in accs:
        pltpu.sync_copy(z_hbm.at[pl.ds(sid * pz, pz)],
                        acc.at[pl.ds(sid * pz, pz)])
    pltpu.sync_copy(n2s_hbm.at[pl.ds(wid * PCH, PCH)], pidx_v)
    plsc.subcore_barrier()
    for hl, acc in zip((h1, h2, h3), accs):
        for c in range(PCH):
            pltpu.sync_copy(hl.at[pl.ds(wid * (PCH * PB) + c * PB, PB)], prows)
            pltpu.sync_copy(prows, acc.at[pidx_v.at[c]], add=True)
    plsc.subcore_barrier()
    for l, acc in enumerate(accs):
        pltpu.sync_copy(acc.at[pl.ds(sid * pz, pz)],
                        sub_hbm.at[cid * L + l, pl.ds(sid * pz, pz)])


_pool_call = functools.partial(
    pl.kernel,
    out_type=jax.ShapeDtypeStruct((NC * L, SA, D), jnp.float32),
    mesh=_mesh,
    scratch_types=[
        pltpu.VMEM((PCH, PB), jnp.int32),
        pltpu.VMEM((PB, D), jnp.float32),
        pltpu.VMEM_SHARED((SA, D), jnp.float32),
        pltpu.VMEM_SHARED((SA, D), jnp.float32),
        pltpu.VMEM_SHARED((SA, D), jnp.float32),
        pltpu.SemaphoreType.DMA,
    ],
)(_pool_body)


def _mlp_body(h_ref, agg_ref, eps_ref, W1_ref, b1_ref, g1_ref, be1_ref,
              W2_ref, b2_ref, g2_ref, be2_ref, out_ref):
    h = h_ref[pl.ds(0, N), :]
    a = agg_ref[0, pl.ds(0, N), :] + agg_ref[1, pl.ds(0, N), :]
    z = (1.0 + eps_ref[0, 0]) * h + a
    z = jnp.dot(z, W1_ref[...], preferred_element_type=jnp.float32) + b1_ref[...]
    m = jnp.mean(z, axis=0, keepdims=True)
    v = jnp.mean((z - m) * (z - m), axis=0, keepdims=True)
    z = (z - m) / jnp.sqrt(v + 1e-5) * g1_ref[...] + be1_ref[...]
    z = jnp.maximum(z, 0.0)
    z = jnp.dot(z, W2_ref[...], preferred_element_type=jnp.float32) + b2_ref[...]
    m = jnp.mean(z, axis=0, keepdims=True)
    v = jnp.mean((z - m) * (z - m), axis=0, keepdims=True)
    z = (z - m) / jnp.sqrt(v + 1e-5) * g2_ref[...] + be2_ref[...]
    z = jnp.maximum(z, 0.0)
    out_ref[pl.ds(0, N), :] = z
    out_ref[pl.ds(N, NP - N), :] = jnp.zeros((NP - N, D), jnp.float32)


_mlp_call = pl.pallas_call(
    _mlp_body,
    out_shape=jax.ShapeDtypeStruct((NP, D), jnp.float32),
)


def _head_body(subp_ref, s2g_ref, lin1W_ref, lin1b_ref,
               bng_ref, bnb_ref, lin2W_ref, lin2b_ref, out_ref):
    s2g = s2g_ref[...]
    oh = (lax.broadcasted_iota(jnp.int32, (G, SA), 0) == s2g).astype(jnp.float32)
    cnt = jnp.sum(oh, axis=1, keepdims=True)
    cnt = jnp.maximum(cnt, 1.0)
    t = lin1b_ref[...]
    for l in range(L):
        sub_l = subp_ref[0, l] + subp_ref[1, l]
        gs_l = jnp.dot(oh, sub_l, preferred_element_type=jnp.float32) / cnt
        t = t + jnp.dot(gs_l, lin1W_ref[pl.ds(l * D, D), :],
                        preferred_element_type=jnp.float32)
    m = jnp.mean(t, axis=0, keepdims=True)
    v = jnp.mean((t - m) * (t - m), axis=0, keepdims=True)
    t = (t - m) / jnp.sqrt(v + 1e-5) * bng_ref[...] + bnb_ref[...]
    t = jnp.maximum(t, 0.0)
    o = jnp.dot(t, lin2W_ref[...], preferred_element_type=jnp.float32) + lin2b_ref[...]
    mx = jnp.max(o, axis=1, keepdims=True)
    lse = jnp.log(jnp.sum(jnp.exp(o - mx), axis=1, keepdims=True))
    out_ref[...] = o - mx - lse


def kernel(x, edge_index, batch, node_to_subgraph, subgraph_to_graph,
           W1, b1, g1, be1, W2, b2, g2, be2, eps,
           lin1_W, lin1_b, bn_g, bn_b, lin2_W, lin2_b):
    C = lin2_W.shape[1]
    f32 = jnp.float32

    src = edge_index[0].astype(jnp.int32)
    dst = edge_index[1].astype(jnp.int32)
    npad = EP - E
    pad_src = N + (jnp.arange(npad, dtype=jnp.int32) % (NP - N))
    pad_dst = N + (jnp.arange(npad, dtype=jnp.int32) % (NA - N))
    src2d = jnp.concatenate([src, pad_src]).reshape(NW * ECH, EPB)
    dst2d = jnp.concatenate([dst, pad_dst]).reshape(NW * ECH, EPB)

    n2s = node_to_subgraph.astype(jnp.int32)
    pad_n2s = jnp.zeros((NP - N,), dtype=jnp.int32)
    n2s2d = jnp.concatenate([n2s, pad_n2s]).reshape(NW * PCH, PB)

    s2g_pad = jnp.concatenate(
        [subgraph_to_graph.astype(jnp.int32),
         jnp.full((SA - S,), G, dtype=jnp.int32)]).reshape(1, SA)

    zeros_np = jnp.zeros((NP, D), f32)
    h = jnp.concatenate([x, jnp.zeros((NP - N, D), f32)], axis=0)

    hs = []
    for l in range(L):
        aggp = _edge_call(h, src2d, dst2d, zeros_np)
        h = _mlp_call(h, aggp, (1.0 * eps[l]).reshape(1, 1),
                      W1[l], b1[l].reshape(1, D), g1[l].reshape(1, D),
                      be1[l].reshape(1, D),
                      W2[l], b2[l].reshape(1, D), g2[l].reshape(1, D),
                      be2[l].reshape(1, D))
        hs.append(h)

    subp = _pool_call(hs[0], hs[1], hs[2], n2s2d, zeros_np)

    head = pl.pallas_call(
        _head_body,
        out_shape=jax.ShapeDtypeStruct((G, C), f32),
    )
    return head(subp.reshape(NC, L, SA, D), s2g_pad,
                lin1_W, lin1_b.reshape(1, D), bn_g.reshape(1, D),
                bn_b.reshape(1, D), lin2_W, lin2_b.reshape(1, C))


# speedup vs baseline: 10.6280x; 1.0061x over previous
import functools

import jax
import jax.numpy as jnp
from jax import lax
from jax.experimental import pallas as pl
from jax.experimental.pallas import tpu as pltpu
from jax.experimental.pallas import tpu_sc as plsc

NC = 2
NS = 16
NW = NC * NS

N = 10000
E = 320000
D = 128
L = 3
S = 2000
G = 64

NP = 10240
NA = 10112
EPB = 40
ECH = 256
IB = 16
NBLK = ECH // IB
NBUF = 6
LAG = 3
EP = NW * ECH * EPB
SA = 2048
PB = 40
PCH = NP // (NW * PB)

_mesh = plsc.VectorSubcoreMesh(
    core_axis_name="c", subcore_axis_name="s", num_cores=NC, num_subcores=NS)


def _edge_body(h_hbm, src_hbm, dst_hbm, z_hbm, agg_hbm,
               src0, src1, dst0, dst1, r0, r1, r2, r3, r4, r5,
               acc_e, g0, g1, g2, g3, g4, g5, s0, s1, s2, s3, s4, s5, isem):
    cid = lax.axis_index("c")
    sid = lax.axis_index("s")
    wid = sid * NC + cid
    ez = NA // NS
    pltpu.sync_copy(z_hbm.at[pl.ds(sid * ez, ez)], acc_e.at[pl.ds(sid * ez, ez)])
    plsc.subcore_barrier()
    srcs = (src0, src1)
    dsts = (dst0, dst1)
    bufs = (r0, r1, r2, r3, r4, r5)
    gsems = (g0, g1, g2, g3, g4, g5)
    ssems = (s0, s1, s2, s3, s4, s5)
    gdesc = [None] * ECH
    sdesc = [None] * ECH
    idesc = [None, None]

    def _scatter(u):
        ub, uj = divmod(u, IB)
        gdesc[u].wait()
        sdesc[u] = pltpu.async_copy(
            bufs[u % NBUF], acc_e.at[dsts[ub % 2].at[uj]],
            ssems[u % NBUF], add=True)

    pltpu.sync_copy(src_hbm.at[pl.ds(wid * ECH, IB)], srcs[0])
    pltpu.sync_copy(dst_hbm.at[pl.ds(wid * ECH, IB)], dsts[0])
    for t in range(ECH):
        blk, j = divmod(t, IB)
        if j == 0 and blk > 0:
            idesc[0].wait()
            idesc[1].wait()
        if j == NBUF and blk + 1 < NBLK:
            nb = blk + 1
            idesc[0] = pltpu.async_copy(
                src_hbm.at[pl.ds(wid * ECH + nb * IB, IB)], srcs[nb % 2], isem)
            idesc[1] = pltpu.async_copy(
                dst_hbm.at[pl.ds(wid * ECH + nb * IB, IB)], dsts[nb % 2], isem)
        if t >= NBUF:
            sdesc[t - NBUF].wait()
        gdesc[t] = pltpu.async_copy(
            h_hbm.at[srcs[blk % 2].at[j]], bufs[t % NBUF], gsems[t % NBUF])
        if t >= LAG:
            _scatter(t - LAG)
    for u in range(ECH - LAG, ECH):
        _scatter(u)
    for u in range(ECH - NBUF, ECH):
        sdesc[u].wait()
    plsc.subcore_barrier()
    pltpu.sync_copy(acc_e.at[pl.ds(sid * ez, ez)],
                    agg_hbm.at[cid, pl.ds(sid * ez, ez)])


_edge_call = functools.partial(
    pl.kernel,
    out_type=jax.ShapeDtypeStruct((NC, NA, D), jnp.float32),
    mesh=_mesh,
    scratch_types=[
        pltpu.VMEM((IB, EPB), jnp.int32),
        pltpu.VMEM((IB, EPB), jnp.int32),
        pltpu.VMEM((IB, EPB), jnp.int32),
        pltpu.VMEM((IB, EPB), jnp.int32),
        pltpu.VMEM((EPB, D), jnp.float32),
        pltpu.VMEM((EPB, D), jnp.float32),
        pltpu.VMEM((EPB, D), jnp.float32),
        pltpu.VMEM((EPB, D), jnp.float32),
        pltpu.VMEM((EPB, D), jnp.float32),
        pltpu.VMEM((EPB, D), jnp.float32),
        pltpu.VMEM_SHARED((NA, D), jnp.float32),
        pltpu.SemaphoreType.DMA,
        pltpu.SemaphoreType.DMA,
        pltpu.SemaphoreType.DMA,
        pltpu.SemaphoreType.DMA,
        pltpu.SemaphoreType.DMA,
        pltpu.SemaphoreType.DMA,
        pltpu.SemaphoreType.DMA,
        pltpu.SemaphoreType.DMA,
        pltpu.SemaphoreType.DMA,
        pltpu.SemaphoreType.DMA,
        pltpu.SemaphoreType.DMA,
        pltpu.SemaphoreType.DMA,
        pltpu.SemaphoreType.DMA,
    ],
)(_edge_body)


def _pool_body(h1, h2, h3, n2s_hbm, z_hbm, sub_hbm,
               pidx_v, prows, acc0, acc1, acc2, sem):
    cid = lax.axis_index("c")
    sid = lax.axis_index("s")
    wid = sid * NC + cid
    pz = SA // NS
    accs = (acc0, acc1, acc2)
    for acc in accs:
        pltpu.sync_copy(z_hbm.at[pl.ds(sid * pz, pz)],
                        acc.at[pl.ds(sid * pz, pz)])
    pltpu.sync_copy(n2s_hbm.at[pl.ds(wid * PCH, PCH)], pidx_v)
    plsc.subcore_barrier()
    for hl, acc in zip((h1, h2, h3), accs):
        for c in range(PCH):
            pltpu.sync_copy(hl.at[pl.ds(wid * (PCH * PB) + c * PB, PB)], prows)
            pltpu.sync_copy(prows, acc.at[pidx_v.at[c]], add=True)
    plsc.subcore_barrier()
    for l, acc in enumerate(accs):
        pltpu.sync_copy(acc.at[pl.ds(sid * pz, pz)],
                        sub_hbm.at[cid * L + l, pl.ds(sid * pz, pz)])


_pool_call = functools.partial(
    pl.kernel,
    out_type=jax.ShapeDtypeStruct((NC * L, SA, D), jnp.float32),
    mesh=_mesh,
    scratch_types=[
        pltpu.VMEM((PCH, PB), jnp.int32),
        pltpu.VMEM((PB, D), jnp.float32),
        pltpu.VMEM_SHARED((SA, D), jnp.float32),
        pltpu.VMEM_SHARED((SA, D), jnp.float32),
        pltpu.VMEM_SHARED((SA, D), jnp.float32),
        pltpu.SemaphoreType.DMA,
    ],
)(_pool_body)


def _mlp_body(h_ref, agg_ref, eps_ref, W1_ref, b1_ref, g1_ref, be1_ref,
              W2_ref, b2_ref, g2_ref, be2_ref, out_ref):
    h = h_ref[pl.ds(0, N), :]
    a = agg_ref[0, pl.ds(0, N), :] + agg_ref[1, pl.ds(0, N), :]
    z = (1.0 + eps_ref[0, 0]) * h + a
    z = jnp.dot(z, W1_ref[...], preferred_element_type=jnp.float32) + b1_ref[...]
    m = jnp.mean(z, axis=0, keepdims=True)
    v = jnp.mean((z - m) * (z - m), axis=0, keepdims=True)
    z = (z - m) / jnp.sqrt(v + 1e-5) * g1_ref[...] + be1_ref[...]
    z = jnp.maximum(z, 0.0)
    z = jnp.dot(z, W2_ref[...], preferred_element_type=jnp.float32) + b2_ref[...]
    m = jnp.mean(z, axis=0, keepdims=True)
    v = jnp.mean((z - m) * (z - m), axis=0, keepdims=True)
    z = (z - m) / jnp.sqrt(v + 1e-5) * g2_ref[...] + be2_ref[...]
    z = jnp.maximum(z, 0.0)
    out_ref[pl.ds(0, N), :] = z
    out_ref[pl.ds(N, NP - N), :] = jnp.zeros((NP - N, D), jnp.float32)


_mlp_call = pl.pallas_call(
    _mlp_body,
    out_shape=jax.ShapeDtypeStruct((NP, D), jnp.float32),
)


def _head_body(subp_ref, s2g_ref, lin1W_ref, lin1b_ref,
               bng_ref, bnb_ref, lin2W_ref, lin2b_ref, out_ref):
    s2g = s2g_ref[...]
    oh = (lax.broadcasted_iota(jnp.int32, (G, SA), 0) == s2g).astype(jnp.float32)
    cnt = jnp.sum(oh, axis=1, keepdims=True)
    cnt = jnp.maximum(cnt, 1.0)
    t = lin1b_ref[...]
    for l in range(L):
        sub_l = subp_ref[0, l] + subp_ref[1, l]
        gs_l = jnp.dot(oh, sub_l, preferred_element_type=jnp.float32) / cnt
        t = t + jnp.dot(gs_l, lin1W_ref[pl.ds(l * D, D), :],
                        preferred_element_type=jnp.float32)
    m = jnp.mean(t, axis=0, keepdims=True)
    v = jnp.mean((t - m) * (t - m), axis=0, keepdims=True)
    t = (t - m) / jnp.sqrt(v + 1e-5) * bng_ref[...] + bnb_ref[...]
    t = jnp.maximum(t, 0.0)
    o = jnp.dot(t, lin2W_ref[...], preferred_element_type=jnp.float32) + lin2b_ref[...]
    mx = jnp.max(o, axis=1, keepdims=True)
    lse = jnp.log(jnp.sum(jnp.exp(o - mx), axis=1, keepdims=True))
    out_ref[...] = o - mx - lse


def kernel(x, edge_index, batch, node_to_subgraph, subgraph_to_graph,
           W1, b1, g1, be1, W2, b2, g2, be2, eps,
           lin1_W, lin1_b, bn_g, bn_b, lin2_W, lin2_b):
    C = lin2_W.shape[1]
    f32 = jnp.float32

    src = edge_index[0].astype(jnp.int32)
    dst = edge_index[1].astype(jnp.int32)
    npad = EP - E
    pad_src = N + (jnp.arange(npad, dtype=jnp.int32) % (NP - N))
    pad_dst = N + (jnp.arange(npad, dtype=jnp.int32) % (NA - N))
    src2d = jnp.concatenate([src, pad_src]).reshape(NW * ECH, EPB)
    dst2d = jnp.concatenate([dst, pad_dst]).reshape(NW * ECH, EPB)

    n2s = node_to_subgraph.astype(jnp.int32)
    pad_n2s = jnp.zeros((NP - N,), dtype=jnp.int32)
    n2s2d = jnp.concatenate([n2s, pad_n2s]).reshape(NW * PCH, PB)

    s2g_pad = jnp.concatenate(
        [subgraph_to_graph.astype(jnp.int32),
         jnp.full((SA - S,), G, dtype=jnp.int32)]).reshape(1, SA)

    zeros_np = jnp.zeros((NP, D), f32)
    h = jnp.concatenate([x, jnp.zeros((NP - N, D), f32)], axis=0)

    hs = []
    for l in range(L):
        aggp = _edge_call(h, src2d, dst2d, zeros_np)
        h = _mlp_call(h, aggp, (1.0 * eps[l]).reshape(1, 1),
                      W1[l], b1[l].reshape(1, D), g1[l].reshape(1, D),
                      be1[l].reshape(1, D),
                      W2[l], b2[l].reshape(1, D), g2[l].reshape(1, D),
                      be2[l].reshape(1, D))
        hs.append(h)

    subp = _pool_call(hs[0], hs[1], hs[2], n2s2d, zeros_np)

    head = pl.pallas_call(
        _head_body,
        out_shape=jax.ShapeDtypeStruct((G, C), f32),
    )
    return head(subp.reshape(NC, L, SA, D), s2g_pad,
                lin1_W, lin1_b.reshape(1, D), bn_g.reshape(1, D),
                bn_b.reshape(1, D), lin2_W, lin2_b.reshape(1, C))


# speedup vs baseline: 11.2803x; 1.0614x over previous
import functools

import jax
import jax.numpy as jnp
from jax import lax
from jax.experimental import pallas as pl
from jax.experimental.pallas import tpu as pltpu
from jax.experimental.pallas import tpu_sc as plsc

NC = 2
NS = 16
NW = NC * NS

N = 10000
E = 320000
D = 128
L = 3
S = 2000
G = 64

NP = 10240
NA = 10112
EPB = 40
ECH = 256
IB = 16
NBLK = ECH // IB
NBUF = 6
LAG = 3
EP = NW * ECH * EPB
SA = 2048
PB = 40
PCH = NP // (NW * PB)

_mesh = plsc.VectorSubcoreMesh(
    core_axis_name="c", subcore_axis_name="s", num_cores=NC, num_subcores=NS)


def _zero_fill(buf, acc, base, rows):
    zv = jnp.zeros((16,), jnp.float32)
    for r in range(40):
        for k in range(D // 16):
            buf[r, pl.ds(k * 16, 16)] = zv
    off = 0
    while off < rows:
        n = min(40, rows - off)
        pltpu.sync_copy(buf.at[pl.ds(0, n)], acc.at[pl.ds(base + off, n)])
        off += n


def _edge_body(h_hbm, src_hbm, dst_hbm, agg_hbm,
               src0, src1, dst0, dst1, r0, r1, r2, r3, r4, r5,
               acc_e, g0, g1, g2, g3, g4, g5, s0, s1, s2, s3, s4, s5, isem):
    cid = lax.axis_index("c")
    sid = lax.axis_index("s")
    wid = sid * NC + cid
    ez = NA // NS
    _zero_fill(r0, acc_e, sid * ez, ez)
    plsc.subcore_barrier()
    srcs = (src0, src1)
    dsts = (dst0, dst1)
    bufs = (r0, r1, r2, r3, r4, r5)
    gsems = (g0, g1, g2, g3, g4, g5)
    ssems = (s0, s1, s2, s3, s4, s5)
    gdesc = [None] * ECH
    sdesc = [None] * ECH
    idesc = [None, None]

    def _scatter(u):
        ub, uj = divmod(u, IB)
        gdesc[u].wait()
        sdesc[u] = pltpu.async_copy(
            bufs[u % NBUF], acc_e.at[dsts[ub % 2].at[uj]],
            ssems[u % NBUF], add=True)

    pltpu.sync_copy(src_hbm.at[pl.ds(wid * ECH, IB)], srcs[0])
    pltpu.sync_copy(dst_hbm.at[pl.ds(wid * ECH, IB)], dsts[0])
    for t in range(ECH):
        blk, j = divmod(t, IB)
        if j == 0 and blk > 0:
            idesc[0].wait()
            idesc[1].wait()
        if j == NBUF and blk + 1 < NBLK:
            nb = blk + 1
            idesc[0] = pltpu.async_copy(
                src_hbm.at[pl.ds(wid * ECH + nb * IB, IB)], srcs[nb % 2], isem)
            idesc[1] = pltpu.async_copy(
                dst_hbm.at[pl.ds(wid * ECH + nb * IB, IB)], dsts[nb % 2], isem)
        if t >= NBUF:
            sdesc[t - NBUF].wait()
        gdesc[t] = pltpu.async_copy(
            h_hbm.at[srcs[blk % 2].at[j]], bufs[t % NBUF], gsems[t % NBUF])
        if t >= LAG:
            _scatter(t - LAG)
    for u in range(ECH - LAG, ECH):
        _scatter(u)
    for u in range(ECH - NBUF, ECH):
        sdesc[u].wait()
    plsc.subcore_barrier()
    pltpu.sync_copy(acc_e.at[pl.ds(sid * ez, ez)],
                    agg_hbm.at[cid, pl.ds(sid * ez, ez)])


_edge_call = functools.partial(
    pl.kernel,
    out_type=jax.ShapeDtypeStruct((NC, NA, D), jnp.float32),
    mesh=_mesh,
    scratch_types=[
        pltpu.VMEM((IB, EPB), jnp.int32),
        pltpu.VMEM((IB, EPB), jnp.int32),
        pltpu.VMEM((IB, EPB), jnp.int32),
        pltpu.VMEM((IB, EPB), jnp.int32),
        pltpu.VMEM((EPB, D), jnp.float32),
        pltpu.VMEM((EPB, D), jnp.float32),
        pltpu.VMEM((EPB, D), jnp.float32),
        pltpu.VMEM((EPB, D), jnp.float32),
        pltpu.VMEM((EPB, D), jnp.float32),
        pltpu.VMEM((EPB, D), jnp.float32),
        pltpu.VMEM_SHARED((NA, D), jnp.float32),
        pltpu.SemaphoreType.DMA,
        pltpu.SemaphoreType.DMA,
        pltpu.SemaphoreType.DMA,
        pltpu.SemaphoreType.DMA,
        pltpu.SemaphoreType.DMA,
        pltpu.SemaphoreType.DMA,
        pltpu.SemaphoreType.DMA,
        pltpu.SemaphoreType.DMA,
        pltpu.SemaphoreType.DMA,
        pltpu.SemaphoreType.DMA,
        pltpu.SemaphoreType.DMA,
        pltpu.SemaphoreType.DMA,
        pltpu.SemaphoreType.DMA,
    ],
)(_edge_body)


def _pool_body(h1, h2, h3, n2s_hbm, sub_hbm,
               pidx_v, p0, p1, p2, p3, acc0, acc1, acc2,
               pg0, pg1, pg2, pg3, ps0, ps1, ps2, ps3):
    cid = lax.axis_index("c")
    sid = lax.axis_index("s")
    wid = sid * NC + cid
    pz = SA // NS
    accs = (acc0, acc1, acc2)
    hls = (h1, h2, h3)
    for i, acc in enumerate(accs):
        _zero_fill(p0, acc, sid * pz, pz)
    pltpu.sync_copy(n2s_hbm.at[pl.ds(wid * PCH, PCH)], pidx_v)
    plsc.subcore_barrier()
    bufs = (p0, p1, p2, p3)
    gsems = (pg0, pg1, pg2, pg3)
    ssems = (ps0, ps1, ps2, ps3)
    TOT = L * PCH
    gdesc = [None] * TOT
    sdesc = [None] * TOT

    def _pscatter(u):
        ul, uc = divmod(u, PCH)
        gdesc[u].wait()
        sdesc[u] = pltpu.async_copy(
            bufs[u % 4], accs[ul].at[pidx_v.at[uc]], ssems[u % 4], add=True)

    for t in range(TOT):
        tl, tc = divmod(t, PCH)
        if t >= 4:
            sdesc[t - 4].wait()
        gdesc[t] = pltpu.async_copy(
            hls[tl].at[pl.ds(wid * (PCH * PB) + tc * PB, PB)],
            bufs[t % 4], gsems[t % 4])
        if t >= 2:
            _pscatter(t - 2)
    for u in range(TOT - 2, TOT):
        _pscatter(u)
    for u in range(TOT - 4, TOT):
        sdesc[u].wait()
    plsc.subcore_barrier()
    for l, acc in enumerate(accs):
        pltpu.sync_copy(acc.at[pl.ds(sid * pz, pz)],
                        sub_hbm.at[cid * L + l, pl.ds(sid * pz, pz)])


_pool_call = functools.partial(
    pl.kernel,
    out_type=jax.ShapeDtypeStruct((NC * L, SA, D), jnp.float32),
    mesh=_mesh,
    scratch_types=[
        pltpu.VMEM((PCH, PB), jnp.int32),
        pltpu.VMEM((PB, D), jnp.float32),
        pltpu.VMEM((PB, D), jnp.float32),
        pltpu.VMEM((PB, D), jnp.float32),
        pltpu.VMEM((PB, D), jnp.float32),
        pltpu.VMEM_SHARED((SA, D), jnp.float32),
        pltpu.VMEM_SHARED((SA, D), jnp.float32),
        pltpu.VMEM_SHARED((SA, D), jnp.float32),
        pltpu.SemaphoreType.DMA,
        pltpu.SemaphoreType.DMA,
        pltpu.SemaphoreType.DMA,
        pltpu.SemaphoreType.DMA,
        pltpu.SemaphoreType.DMA,
        pltpu.SemaphoreType.DMA,
        pltpu.SemaphoreType.DMA,
        pltpu.SemaphoreType.DMA,
    ],
)(_pool_body)


def _mlp_body(h_ref, agg_ref, eps_ref, W1_ref, b1_ref, g1_ref, be1_ref,
              W2_ref, b2_ref, g2_ref, be2_ref, out_ref):
    h = h_ref[pl.ds(0, N), :]
    a = agg_ref[0, pl.ds(0, N), :] + agg_ref[1, pl.ds(0, N), :]
    z = (1.0 + eps_ref[0, 0]) * h + a
    z = jnp.dot(z, W1_ref[...], preferred_element_type=jnp.float32) + b1_ref[...]
    m = jnp.mean(z, axis=0, keepdims=True)
    v = jnp.mean((z - m) * (z - m), axis=0, keepdims=True)
    z = (z - m) / jnp.sqrt(v + 1e-5) * g1_ref[...] + be1_ref[...]
    z = jnp.maximum(z, 0.0)
    z = jnp.dot(z, W2_ref[...], preferred_element_type=jnp.float32) + b2_ref[...]
    m = jnp.mean(z, axis=0, keepdims=True)
    v = jnp.mean((z - m) * (z - m), axis=0, keepdims=True)
    z = (z - m) / jnp.sqrt(v + 1e-5) * g2_ref[...] + be2_ref[...]
    z = jnp.maximum(z, 0.0)
    out_ref[pl.ds(0, N), :] = z
    out_ref[pl.ds(N, NP - N), :] = jnp.zeros((NP - N, D), jnp.float32)


_mlp_call = pl.pallas_call(
    _mlp_body,
    out_shape=jax.ShapeDtypeStruct((NP, D), jnp.float32),
)


def _head_body(subp_ref, s2g_ref, lin1W_ref, lin1b_ref,
               bng_ref, bnb_ref, lin2W_ref, lin2b_ref, out_ref):
    s2g = s2g_ref[...]
    oh = (lax.broadcasted_iota(jnp.int32, (G, SA), 0) == s2g).astype(jnp.float32)
    cnt = jnp.sum(oh, axis=1, keepdims=True)
    cnt = jnp.maximum(cnt, 1.0)
    t = lin1b_ref[...]
    for l in range(L):
        sub_l = subp_ref[0, l] + subp_ref[1, l]
        gs_l = jnp.dot(oh, sub_l, preferred_element_type=jnp.float32) / cnt
        t = t + jnp.dot(gs_l, lin1W_ref[pl.ds(l * D, D), :],
                        preferred_element_type=jnp.float32)
    m = jnp.mean(t, axis=0, keepdims=True)
    v = jnp.mean((t - m) * (t - m), axis=0, keepdims=True)
    t = (t - m) / jnp.sqrt(v + 1e-5) * bng_ref[...] + bnb_ref[...]
    t = jnp.maximum(t, 0.0)
    o = jnp.dot(t, lin2W_ref[...], preferred_element_type=jnp.float32) + lin2b_ref[...]
    mx = jnp.max(o, axis=1, keepdims=True)
    lse = jnp.log(jnp.sum(jnp.exp(o - mx), axis=1, keepdims=True))
    out_ref[...] = o - mx - lse


def kernel(x, edge_index, batch, node_to_subgraph, subgraph_to_graph,
           W1, b1, g1, be1, W2, b2, g2, be2, eps,
           lin1_W, lin1_b, bn_g, bn_b, lin2_W, lin2_b):
    C = lin2_W.shape[1]
    f32 = jnp.float32

    src = edge_index[0].astype(jnp.int32)
    dst = edge_index[1].astype(jnp.int32)
    npad = EP - E
    pad_src = N + (jnp.arange(npad, dtype=jnp.int32) % (NP - N))
    pad_dst = N + (jnp.arange(npad, dtype=jnp.int32) % (NA - N))
    src2d = jnp.concatenate([src, pad_src]).reshape(NW * ECH, EPB)
    dst2d = jnp.concatenate([dst, pad_dst]).reshape(NW * ECH, EPB)

    n2s = node_to_subgraph.astype(jnp.int32)
    pad_n2s = jnp.zeros((NP - N,), dtype=jnp.int32)
    n2s2d = jnp.concatenate([n2s, pad_n2s]).reshape(NW * PCH, PB)

    s2g_pad = jnp.concatenate(
        [subgraph_to_graph.astype(jnp.int32),
         jnp.full((SA - S,), G, dtype=jnp.int32)]).reshape(1, SA)

    h = jnp.concatenate([x, jnp.zeros((NP - N, D), f32)], axis=0)

    hs = []
    for l in range(L):
        aggp = _edge_call(h, src2d, dst2d)
        h = _mlp_call(h, aggp, (1.0 * eps[l]).reshape(1, 1),
                      W1[l], b1[l].reshape(1, D), g1[l].reshape(1, D),
                      be1[l].reshape(1, D),
                      W2[l], b2[l].reshape(1, D), g2[l].reshape(1, D),
                      be2[l].reshape(1, D))
        hs.append(h)

    subp = _pool_call(hs[0], hs[1], hs[2], n2s2d)

    head = pl.pallas_call(
        _head_body,
        out_shape=jax.ShapeDtypeStruct((G, C), f32),
    )
    return head(subp.reshape(NC, L, SA, D), s2g_pad,
                lin1_W, lin1_b.reshape(1, D), bn_g.reshape(1, D),
                bn_b.reshape(1, D), lin2_W, lin2_b.reshape(1, C))
